# baseline (speedup 1.0000x reference)
import functools

import jax
import jax.numpy as jnp
from jax import lax
from jax.experimental import pallas as pl
from jax.experimental.pallas import tpu as pltpu
from jax.experimental.pallas import tpu_sc as plsc

N = 10000
NP = 10240
E = 320000
D = 128
MIN_NORM = 1e-15

NS = 16
K = 56
ROWS_PER_TILE = NP // NS
EDGES_PER_TILE = E // NS
EPTP = 20160
NCHUNK = EPTP // K
WB = 40
WCHUNK = ROWS_PER_TILE // WB
NBUF = 4


def _sc_segment_means(x, src_all, dst_all, zeros_kd, zeros_np):
  mesh = plsc.VectorSubcoreMesh(core_axis_name="c", subcore_axis_name="s")

  @functools.partial(
      pl.kernel,
      out_type=[
          jax.ShapeDtypeStruct((2, NP, D), jnp.float32),
          jax.ShapeDtypeStruct((2, NP), jnp.float32),
      ],
      mesh=mesh,
      compiler_params=pltpu.CompilerParams(needs_layout_passes=False),
      scratch_types=(
          [pltpu.VMEM((K,), jnp.int32) for _ in range(NBUF)] +
          [pltpu.VMEM((K,), jnp.int32) for _ in range(NBUF)] +
          [pltpu.VMEM((K, D), jnp.float32) for _ in range(NBUF)] +
          [
              pltpu.VMEM((NP,), jnp.float32),
              pltpu.VMEM((NS, 128), jnp.float32),
              pltpu.VMEM_SHARED((NP, D), jnp.float32),
              pltpu.VMEM_SHARED((NS, NP // 4), jnp.float32),
          ] +
          [pltpu.SemaphoreType.DMA for _ in range(4 * NBUF)]
      ),
  )
  def sc_kernel(x_hbm, src_hbm, dst_hbm, zkd_hbm, znp_hbm, agg_out, cnt_out,
                *refs):
    src_bufs = refs[0:NBUF]
    dst_bufs = refs[NBUF:2 * NBUF]
    row_bufs = refs[2 * NBUF:3 * NBUF]
    hist_v, hsum_v, acc_sh, hist_sh = refs[3 * NBUF:3 * NBUF + 4]
    sems = refs[3 * NBUF + 4:]
    semg = sems[0:NBUF]
    semsc = sems[NBUF:2 * NBUF]
    semis = sems[2 * NBUF:3 * NBUF]
    semid = sems[3 * NBUF:4 * NBUF]
    c = lax.axis_index("c")
    s = lax.axis_index("s")
    r0 = s * ROWS_PER_TILE
    ebase = (c * NS + s) * EPTP

    def start_idx(b, chunk):
      off = ebase + chunk * K
      pltpu.async_copy(src_hbm.at[pl.ds(off, K)], src_bufs[b], semis[b])
      pltpu.async_copy(dst_hbm.at[pl.ds(off, K)], dst_bufs[b], semid[b])

    def wait_idx(b):
      pltpu.make_async_copy(src_hbm.at[pl.ds(0, K)], src_bufs[b],
                            semis[b]).wait()
      pltpu.make_async_copy(dst_hbm.at[pl.ds(0, K)], dst_bufs[b],
                            semid[b]).wait()

    def start_gather(b):
      pltpu.async_copy(x_hbm.at[src_bufs[b]], row_bufs[b], semg[b])

    def wait_gather(b):
      pltpu.make_async_copy(x_hbm.at[src_bufs[b]], row_bufs[b],
                            semg[b]).wait()

    def start_scatter(b):
      pltpu.async_copy(row_bufs[b], acc_sh.at[dst_bufs[b]], semsc[b],
                       add=True)

    def wait_scatter(b):
      pltpu.make_async_copy(row_bufs[b], acc_sh.at[dst_bufs[b]],
                            semsc[b]).wait()

    def hist_update(b):
      ones16 = jnp.full((16,), 1.0, jnp.float32)
      for kk in range(K // 16):
        idx16 = dst_bufs[b][pl.ds(kk * 16, 16)]
        plsc.addupdate_scatter(hist_v, [idx16], ones16)
      if K % 16:
        tail = dst_bufs[b][pl.ds(K - 16, 16)]
        keep = lax.iota(jnp.int32, 16) >= (16 - K % 16)
        plsc.addupdate_scatter(hist_v, [tail], ones16, mask=keep)

    def step(cch, b, do_scwait, do_idx, do_gather):
      if do_gather:
        wait_idx((b + 1) % NBUF)
        start_gather((b + 1) % NBUF)
      wait_gather(b)
      hist_update(b)
      start_scatter(b)
      if do_scwait:
        wait_scatter((b + 2) % NBUF)
      if do_idx:
        start_idx((b + 2) % NBUF, cch + 2)

    start_idx(0, 0)
    start_idx(1, 1)
    pltpu.sync_copy(zkd_hbm, row_bufs[3])
    pltpu.sync_copy(znp_hbm, hist_v)
    for j in range(WCHUNK):
      pltpu.sync_copy(row_bufs[3].at[pl.ds(0, WB)],
                      acc_sh.at[pl.ds(r0 + j * WB, WB)])
    wait_idx(0)
    start_gather(0)
    plsc.subcore_barrier()

    step(0, 0, False, True, True)
    step(1, 1, False, True, True)

    def body(p, carry):
      cch = 4 * p + 2
      for q in range(4):
        step(cch + q, (2 + q) % NBUF, True, True, True)
      return carry

    lax.fori_loop(0, (NCHUNK - 4) // 4, body, 0)
    step(NCHUNK - 2, 2, True, False, True)
    step(NCHUNK - 1, 3, True, False, False)
    wait_scatter(2)
    wait_scatter(3)

    QHALF = NP // 4
    for h in range(4):
      pltpu.sync_copy(hist_v.at[pl.ds(h * QHALF, QHALF)], hist_sh.at[s])
      plsc.subcore_barrier()

      @pl.when(s // (NS // 4) == h)
      def _():
        QTR = 128
        for q in range(ROWS_PER_TILE // QTR):
          for t in range(NS):
            pltpu.sync_copy(
                hist_sh.at[t, pl.ds(r0 - h * QHALF + q * QTR, QTR)],
                hsum_v.at[t])
          for cc in range(QTR // 16):
            sl = pl.ds(cc * 16, 16)
            tot = hsum_v[0, sl]
            for t in range(1, NS):
              tot = tot + hsum_v[t, sl]
            hsum_v[0, sl] = tot
          pltpu.sync_copy(hsum_v.at[0],
                          cnt_out.at[c, pl.ds(r0 + q * QTR, QTR)])
      plsc.subcore_barrier()

    for j in range(WCHUNK):
      sl = pl.ds(r0 + j * WB, WB)
      pltpu.sync_copy(acc_sh.at[sl], row_bufs[j % 2].at[pl.ds(0, WB)])
      pltpu.sync_copy(row_bufs[j % 2].at[pl.ds(0, WB)], agg_out.at[c, sl])

  return sc_kernel(x, src_all, dst_all, zeros_kd, zeros_np)


def _rownorm(v):
  return jnp.sqrt(jnp.sum(v * v, axis=-1, keepdims=True))


def _artanh(v):
  v = jnp.clip(v, -1.0 + 1e-7, 1.0 - 1e-7)
  return 0.5 * jnp.log((1.0 + v) / (1.0 - v))


def _proj(v):
  norm = jnp.maximum(_rownorm(v), MIN_NORM)
  maxnorm = 1.0 - 1e-5
  return jnp.where(norm > maxnorm, v / norm * maxnorm, v)


def _mobius_matvec(W, v):
  x_norm = jnp.maximum(_rownorm(v), MIN_NORM)
  mx = lax.dot_general(v, W, (((1,), (1,)), ((), ())),
                       preferred_element_type=jnp.float32)
  mx_norm = jnp.maximum(_rownorm(mx), MIN_NORM)
  res = jnp.tanh(mx_norm / x_norm * _artanh(x_norm)) * mx / mx_norm
  allzero = jnp.max(jnp.abs(mx), axis=-1, keepdims=True) == 0.0
  return jnp.where(allzero, 0.0, res)


def _mobius_add(a, b):
  a2 = jnp.sum(a * a, axis=-1, keepdims=True)
  b2 = jnp.sum(b * b, axis=-1, keepdims=True)
  ab = jnp.sum(a * b, axis=-1, keepdims=True)
  num = (1.0 + 2.0 * ab + b2) * a + (1.0 - a2) * b
  den = 1.0 + 2.0 * ab + a2 * b2
  return num / jnp.maximum(den, MIN_NORM)


def _expmap0(u):
  u_norm = jnp.maximum(_rownorm(u), MIN_NORM)
  return jnp.tanh(u_norm) * u / u_norm


def _cc_body(x_ref, wpc_ref, bp_ref, wnc_ref, bn_ref, o_ref):
  xb = x_ref[...]

  def half(wcc_ref, b_ref):
    hyp_b = _proj(_expmap0(b_ref[...]))
    res = _proj(_mobius_matvec(wcc_ref[...], xb))
    return _proj(_mobius_add(res, hyp_b))

  o_ref[:, :D] = half(wpc_ref, bp_ref)
  o_ref[:, D:] = half(wnc_ref, bn_ref)


def _tc_cc(x, W_pos_cc, b_pos_cc, W_neg_cc, b_neg_cc):
  B = 1000
  grid = (N // B,)
  row_spec = lambda w: pl.BlockSpec((B, w), lambda i: (i, 0))
  full_spec = pl.BlockSpec((D, D), lambda i: (0, 0))
  bias_spec = pl.BlockSpec((1, D), lambda i: (0, 0))
  return pl.pallas_call(
      _cc_body,
      grid=grid,
      in_specs=[row_spec(D), full_spec, bias_spec, full_spec, bias_spec],
      out_specs=row_spec(2 * D),
      out_shape=jax.ShapeDtypeStruct((N, 2 * D), jnp.float32),
  )(x, W_pos_cc, b_pos_cc.reshape(1, D), W_neg_cc, b_neg_cc.reshape(1, D))


def _agg_body(ap_ref, cp_ref, an_ref, cn_ref, wp_ref, wn_ref, cc_ref, o_ref):
  def half(agg_ref, cnt_ref, w_ref):
    mean = agg_ref[...] / jnp.maximum(cnt_ref[...], 1.0)
    return _proj(_mobius_matvec(w_ref[...], mean))

  ccb = cc_ref[...]
  o_ref[:, :D] = half(ap_ref, cp_ref, wp_ref) + ccb[:, :D]
  o_ref[:, D:] = half(an_ref, cn_ref, wn_ref) + ccb[:, D:]


def _tc_agg(agg, cnt, W_pos, W_neg, cc_out):
  B = 1000
  grid = (N // B,)
  row_spec = lambda w: pl.BlockSpec((B, w), lambda i: (i, 0))
  full_spec = pl.BlockSpec((D, D), lambda i: (0, 0))
  return pl.pallas_call(
      _agg_body,
      grid=grid,
      in_specs=[row_spec(D), row_spec(1), row_spec(D), row_spec(1),
                full_spec, full_spec, row_spec(2 * D)],
      out_specs=row_spec(2 * D),
      out_shape=jax.ShapeDtypeStruct((N, 2 * D), jnp.float32),
  )(agg[0], cnt[0].reshape(NP, 1), agg[1], cnt[1].reshape(NP, 1),
    W_pos, W_neg, cc_out)


def kernel(x, pos_edge_index, neg_edge_index,
           W_pos, W_pos_cc, b_pos_cc, W_neg, W_neg_cc, b_neg_cc):
  npad = EPTP - EDGES_PER_TILE
  def lay(a, fill):
    a = a.astype(jnp.int32).reshape(2 * NS, EDGES_PER_TILE)
    return jnp.pad(a, ((0, 0), (0, npad)), constant_values=fill).reshape(-1)
  src_all = lay(jnp.concatenate([pos_edge_index[0], neg_edge_index[0]]), 0)
  dst_all = lay(jnp.concatenate([pos_edge_index[1], neg_edge_index[1]]),
                NP - 1)
  zeros_kd = jnp.zeros((K, D), jnp.float32)
  zeros_np = jnp.zeros((NP,), jnp.float32)
  agg, cnt = _sc_segment_means(x, src_all, dst_all, zeros_kd, zeros_np)
  cc_out = _tc_cc(x, W_pos_cc, b_pos_cc, W_neg_cc, b_neg_cc)
  return _tc_agg(agg, cnt, W_pos, W_neg, cc_out)


# speedup vs baseline: 1.5090x; 1.5090x over previous
import functools

import jax
import jax.numpy as jnp
from jax import lax
from jax.experimental import pallas as pl
from jax.experimental.pallas import tpu as pltpu
from jax.experimental.pallas import tpu_sc as plsc

N = 10000
NP = 10240
E = 320000
D = 128
MIN_NORM = 1e-15

NS = 16
K = 80
ROWS_PER_TILE = NP // NS
EDGES_PER_TILE = E // NS
NCHUNK = EDGES_PER_TILE // K
WCHUNK = ROWS_PER_TILE // K


def _sc_segment_means(x, src_all, dst_all, zeros_kd, zeros_np):
  mesh = plsc.VectorSubcoreMesh(core_axis_name="c", subcore_axis_name="s")

  @functools.partial(
      pl.kernel,
      out_type=[
          jax.ShapeDtypeStruct((2, NP, D), jnp.float32),
          jax.ShapeDtypeStruct((2, NP), jnp.float32),
      ],
      mesh=mesh,
      compiler_params=pltpu.CompilerParams(needs_layout_passes=False),
      scratch_types=[
          pltpu.VMEM((K,), jnp.int32),
          pltpu.VMEM((K,), jnp.int32),
          pltpu.VMEM((K,), jnp.int32),
          pltpu.VMEM((K,), jnp.int32),
          pltpu.VMEM((K, D), jnp.float32),
          pltpu.VMEM((K, D), jnp.float32),
          pltpu.VMEM((NP,), jnp.float32),
          pltpu.VMEM((NS, 128), jnp.float32),
          pltpu.VMEM((ROWS_PER_TILE,), jnp.float32),
          pltpu.VMEM_SHARED((NP, D), jnp.float32),
          pltpu.VMEM_SHARED((NS, NP // 2), jnp.float32),
          pltpu.SemaphoreType.DMA,
          pltpu.SemaphoreType.DMA,
          pltpu.SemaphoreType.DMA,
          pltpu.SemaphoreType.DMA,
          pltpu.SemaphoreType.DMA,
          pltpu.SemaphoreType.DMA,
      ],
  )
  def sc_kernel(x_hbm, src_hbm, dst_hbm, zkd_hbm, znp_hbm, agg_out, cnt_out,
                src_v0, dst_v0, src_v1, dst_v1, rows_v0, rows_v1,
                hist_v, hsum_v, cnt_v, acc_sh, hist_sh,
                semg0, semg1, semis0, semid0, semis1, semid1):
    c = lax.axis_index("c")
    s = lax.axis_index("s")
    r0 = s * ROWS_PER_TILE
    ebase = c * E + s * EDGES_PER_TILE
    bufs = ((src_v0, dst_v0, rows_v0, semg0, semis0, semid0),
            (src_v1, dst_v1, rows_v1, semg1, semis1, semid1))

    def start_idx(bi, chunk):
      sv, dv, _, _, sis, sid = bufs[bi]
      b = ebase + chunk * K
      pltpu.async_copy(src_hbm.at[pl.ds(b, K)], sv, sis)
      pltpu.async_copy(dst_hbm.at[pl.ds(b, K)], dv, sid)

    def wait_idx(bi):
      sv, dv, _, _, sis, sid = bufs[bi]
      pltpu.make_async_copy(src_hbm.at[pl.ds(0, K)], sv, sis).wait()
      pltpu.make_async_copy(dst_hbm.at[pl.ds(0, K)], dv, sid).wait()

    def start_gather(bi):
      sv, _, rv, sg, _, _ = bufs[bi]
      pltpu.async_copy(x_hbm.at[sv], rv, sg)

    def wait_gather(bi):
      sv, _, rv, sg, _, _ = bufs[bi]
      pltpu.make_async_copy(x_hbm.at[sv], rv, sg).wait()

    def consume(bi):
      _, dv, rv, _, _, _ = bufs[bi]
      ones16 = jnp.full((16,), 1.0, jnp.float32)
      for kk in range(K // 16):
        idx16 = dv[pl.ds(kk * 16, 16)]
        plsc.addupdate_scatter(hist_v, [idx16], ones16)
      pltpu.sync_copy(rv, acc_sh.at[dv], add=True)

    start_idx(0, 0)
    start_idx(1, 1)
    pltpu.sync_copy(zkd_hbm, rows_v1)
    pltpu.sync_copy(znp_hbm, hist_v)
    for j in range(WCHUNK):
      pltpu.sync_copy(rows_v1, acc_sh.at[pl.ds(r0 + j * K, K)])
    wait_idx(0)
    start_gather(0)
    plsc.subcore_barrier()

    def body(p, carry):
      wait_gather(0)
      wait_idx(1)
      start_gather(1)
      consume(0)
      start_idx(0, 2 * p + 2)
      wait_gather(1)
      wait_idx(0)
      start_gather(0)
      consume(1)
      start_idx(1, 2 * p + 3)
      return carry

    lax.fori_loop(0, NCHUNK // 2, body, 0)
    wait_gather(0)
    wait_idx(1)

    HALF = NP // 2
    for h in range(2):
      pltpu.sync_copy(hist_v.at[pl.ds(h * HALF, HALF)], hist_sh.at[s])
      plsc.subcore_barrier()

      @pl.when(s // (NS // 2) == h)
      def _():
        QTR = 128
        for q in range(ROWS_PER_TILE // QTR):
          for t in range(NS):
            pltpu.sync_copy(
                hist_sh.at[t, pl.ds(r0 - h * HALF + q * QTR, QTR)],
                hsum_v.at[t])
          for cc in range(QTR // 16):
            sl = pl.ds(cc * 16, 16)
            tot = hsum_v[0, sl]
            for t in range(1, NS):
              tot = tot + hsum_v[t, sl]
            cnt_v[pl.ds(q * QTR + cc * 16, 16)] = tot
        pltpu.sync_copy(cnt_v, cnt_out.at[c, pl.ds(r0, ROWS_PER_TILE)])
      plsc.subcore_barrier()

    for j in range(WCHUNK):
      sl = pl.ds(r0 + j * K, K)
      pltpu.sync_copy(acc_sh.at[sl], rows_v0)
      pltpu.sync_copy(rows_v0, agg_out.at[c, sl])

  return sc_kernel(x, src_all, dst_all, zeros_kd, zeros_np)


def _rownorm(v):
  return jnp.sqrt(jnp.sum(v * v, axis=-1, keepdims=True))


def _artanh(v):
  v = jnp.clip(v, -1.0 + 1e-7, 1.0 - 1e-7)
  return 0.5 * jnp.log((1.0 + v) / (1.0 - v))


def _proj(v):
  norm = jnp.maximum(_rownorm(v), MIN_NORM)
  maxnorm = 1.0 - 1e-5
  return jnp.where(norm > maxnorm, v / norm * maxnorm, v)


def _mobius_matvec(W, v):
  x_norm = jnp.maximum(_rownorm(v), MIN_NORM)
  mx = lax.dot_general(v, W, (((1,), (1,)), ((), ())),
                       preferred_element_type=jnp.float32)
  mx_norm = jnp.maximum(_rownorm(mx), MIN_NORM)
  res = jnp.tanh(mx_norm / x_norm * _artanh(x_norm)) * mx / mx_norm
  allzero = jnp.max(jnp.abs(mx), axis=-1, keepdims=True) == 0.0
  return jnp.where(allzero, 0.0, res)


def _mobius_add(a, b):
  a2 = jnp.sum(a * a, axis=-1, keepdims=True)
  b2 = jnp.sum(b * b, axis=-1, keepdims=True)
  ab = jnp.sum(a * b, axis=-1, keepdims=True)
  num = (1.0 + 2.0 * ab + b2) * a + (1.0 - a2) * b
  den = 1.0 + 2.0 * ab + a2 * b2
  return num / jnp.maximum(den, MIN_NORM)


def _expmap0(u):
  u_norm = jnp.maximum(_rownorm(u), MIN_NORM)
  return jnp.tanh(u_norm) * u / u_norm


def _cc_body(x_ref, wpc_ref, bp_ref, wnc_ref, bn_ref, o_ref):
  xb = x_ref[...]

  def half(wcc_ref, b_ref):
    hyp_b = _proj(_expmap0(b_ref[...]))
    res = _proj(_mobius_matvec(wcc_ref[...], xb))
    return _proj(_mobius_add(res, hyp_b))

  o_ref[:, :D] = half(wpc_ref, bp_ref)
  o_ref[:, D:] = half(wnc_ref, bn_ref)


def _tc_cc(x, W_pos_cc, b_pos_cc, W_neg_cc, b_neg_cc):
  B = 1000
  grid = (N // B,)
  row_spec = lambda w: pl.BlockSpec((B, w), lambda i: (i, 0))
  full_spec = pl.BlockSpec((D, D), lambda i: (0, 0))
  bias_spec = pl.BlockSpec((1, D), lambda i: (0, 0))
  return pl.pallas_call(
      _cc_body,
      grid=grid,
      in_specs=[row_spec(D), full_spec, bias_spec, full_spec, bias_spec],
      out_specs=row_spec(2 * D),
      out_shape=jax.ShapeDtypeStruct((N, 2 * D), jnp.float32),
  )(x, W_pos_cc, b_pos_cc.reshape(1, D), W_neg_cc, b_neg_cc.reshape(1, D))


def _agg_body(ap_ref, cp_ref, an_ref, cn_ref, wp_ref, wn_ref, cc_ref, o_ref):
  def half(agg_ref, cnt_ref, w_ref):
    mean = agg_ref[...] / jnp.maximum(cnt_ref[...], 1.0)
    return _proj(_mobius_matvec(w_ref[...], mean))

  ccb = cc_ref[...]
  o_ref[:, :D] = half(ap_ref, cp_ref, wp_ref) + ccb[:, :D]
  o_ref[:, D:] = half(an_ref, cn_ref, wn_ref) + ccb[:, D:]


def _tc_agg(agg, cnt, W_pos, W_neg, cc_out):
  B = 1000
  grid = (N // B,)
  row_spec = lambda w: pl.BlockSpec((B, w), lambda i: (i, 0))
  full_spec = pl.BlockSpec((D, D), lambda i: (0, 0))
  return pl.pallas_call(
      _agg_body,
      grid=grid,
      in_specs=[row_spec(D), row_spec(1), row_spec(D), row_spec(1),
                full_spec, full_spec, row_spec(2 * D)],
      out_specs=row_spec(2 * D),
      out_shape=jax.ShapeDtypeStruct((N, 2 * D), jnp.float32),
  )(agg[0], cnt[0].reshape(NP, 1), agg[1], cnt[1].reshape(NP, 1),
    W_pos, W_neg, cc_out)


def kernel(x, pos_edge_index, neg_edge_index,
           W_pos, W_pos_cc, b_pos_cc, W_neg, W_neg_cc, b_neg_cc):
  pad = jnp.zeros((2 * K,), jnp.int32)
  src_all = jnp.concatenate(
      [pos_edge_index[0], neg_edge_index[0], pad]).astype(jnp.int32)
  dst_all = jnp.concatenate(
      [pos_edge_index[1], neg_edge_index[1], pad]).astype(jnp.int32)
  zeros_kd = jnp.zeros((K, D), jnp.float32)
  zeros_np = jnp.zeros((NP,), jnp.float32)
  agg, cnt = _sc_segment_means(x, src_all, dst_all, zeros_kd, zeros_np)
  cc_out = _tc_cc(x, W_pos_cc, b_pos_cc, W_neg_cc, b_neg_cc)
  return _tc_agg(agg, cnt, W_pos, W_neg, cc_out)
